# Initial kernel scaffold; baseline (speedup 1.0000x reference)
import jax
import jax.numpy as jnp
from jax.experimental import pallas as pl


def kernel(obs_times, event_pt, sample_idx, X, M, batch_idx, device, T, W1, b1, W2, b2, Wr, br, Ws, bs, Wst, bst):
    raise NotImplementedError("write your pallas kernel here")


# speedup vs baseline: 15.7371x; 15.7371x over previous
import math

import jax
import jax.numpy as jnp
from jax import lax
from jax.experimental import pallas as pl
from jax.experimental.pallas import tpu as pltpu

H = 512
I = 128
S = 7
B = 8
L = 2048

LOG10_HALF = math.log(10.0) / 2.0
LOG_TAU = [s * LOG10_HALF for s in range(S)]
TAU = [math.exp(v) for v in LOG_TAU]


def _pick_row(blk, rem):
    rowmask = lax.broadcasted_iota(jnp.int32, blk.shape, 0) == rem
    return jnp.sum(jnp.where(rowmask, blk, 0.0), axis=0, keepdims=True)


def _ctgru_seq_kernel(
    bidx_ref,
    x_ref,
    m_ref,
    w1t_ref,
    b1_ref,
    w2t_ref,
    b2_ref,
    wr_x_ref,
    wr_h_ref,
    br_ref,
    ws_x_ref,
    ws_h_ref,
    bs_ref,
    wt_x_ref,
    wt_h_ref,
    bt_ref,
    loss_ref,
    ratio_ref,
    state_ref,
    lastt_ref,
    acc_ref,
):
    state_ref[...] = jnp.zeros((B * 8, H), jnp.float32)
    acc_ref[...] = jnp.zeros((1, I), jnp.float32)
    for b in range(B):
        lastt_ref[b] = 0.0

    def step(t, carry):
        idx = bidx_ref[t]
        tbase = (t // 8) * 8
        trem = t - tbase
        x = _pick_row(x_ref[pl.ds(tbase, 8), :], trem)
        m = _pick_row(m_ref[pl.ds(tbase, 8), :], trem)

        blk = state_ref[pl.ds(idx * 8, 8), :]
        h_row = blk[S:S + 1, :]

        a = jnp.maximum(
            jnp.dot(h_row, w1t_ref[...], preferred_element_type=jnp.float32)
            + b1_ref[...], 0.0)
        p = jnp.dot(a, w2t_ref[...], preferred_element_type=jnp.float32) + b2_ref[...]
        acc_ref[...] += jnp.abs(x - p) * m

        rr = (jnp.dot(x, wr_x_ref[...], preferred_element_type=jnp.float32)
              + jnp.dot(h_row, wr_h_ref[...], preferred_element_type=jnp.float32)
              + br_ref[...])
        q = [-jnp.square(rr[:, s * H:(s + 1) * H] - LOG_TAU[s]) for s in range(S)]
        mx = q[0]
        for s in range(1, S):
            mx = jnp.maximum(mx, q[s])
        e = [jnp.exp(q[s] - mx) for s in range(S)]
        den = e[0]
        for s in range(1, S):
            den = den + e[s]
        rsum = e[0] * blk[0:1, :]
        for s in range(1, S):
            rsum += e[s] * blk[s:s + 1, :]
        rsum = rsum / den

        h_tilde = jnp.tanh(
            jnp.dot(x, ws_x_ref[...], preferred_element_type=jnp.float32)
            + jnp.dot(rsum, ws_h_ref[...], preferred_element_type=jnp.float32)
            + bs_ref[...])

        zz = (jnp.dot(x, wt_x_ref[...], preferred_element_type=jnp.float32)
              + jnp.dot(h_row, wt_h_ref[...], preferred_element_type=jnp.float32)
              + bt_ref[...])
        qz = [-jnp.square(zz[:, s * H:(s + 1) * H] - LOG_TAU[s]) for s in range(S)]
        mz = qz[0]
        for s in range(1, S):
            mz = jnp.maximum(mz, qz[s])
        ez = [jnp.exp(qz[s] - mz) for s in range(S)]
        dz = ez[0]
        for s in range(1, S):
            dz = dz + ez[s]

        ot = t.astype(jnp.float32)
        interval = ot - lastt_ref[idx]
        lastt_ref[idx] = ot

        new_h = blk[0:1, :]
        for s in range(1, S):
            new_h = new_h + blk[s:s + 1, :]
        new_rows = []
        for s in range(S):
            z_s = ez[s] / dz
            expf = jnp.exp(-interval / TAU[s])
            new_rows.append(((1.0 - z_s) * blk[s:s + 1, :] + z_s * h_tilde) * expf)
        new_rows.append(new_h)
        state_ref[pl.ds(idx * 8, 8), :] = jnp.concatenate(new_rows, axis=0)
        return carry

    lax.fori_loop(0, L, step, 0)

    loss = jnp.sum(acc_ref[...])
    tot_m = jnp.sum(m_ref[...])
    loss_ref[0] = loss
    ratio_ref[0] = loss / tot_m


def kernel(obs_times, event_pt, sample_idx, X, M, batch_idx, device, T,
           W1, b1, W2, b2, Wr, br, Ws, bs, Wst, bst):
    def split_sh(W):
        Wp = W.reshape(H, S, I + H).transpose(2, 1, 0).reshape(I + H, S * H)
        return Wp[:I], Wp[I:]

    wr_x, wr_h = split_sh(Wr)
    wt_x, wt_h = split_sh(Wst)
    br_p = br.reshape(H, S).T.reshape(1, S * H)
    bt_p = bst.reshape(H, S).T.reshape(1, S * H)
    out = pl.pallas_call(
        _ctgru_seq_kernel,
        out_shape=(
            jax.ShapeDtypeStruct((1,), jnp.float32),
            jax.ShapeDtypeStruct((1,), jnp.float32),
        ),
        in_specs=[
            pl.BlockSpec(memory_space=pltpu.SMEM),
        ] + [
            pl.BlockSpec(memory_space=pltpu.VMEM) for _ in range(15)
        ],
        out_specs=(
            pl.BlockSpec(memory_space=pltpu.SMEM),
            pl.BlockSpec(memory_space=pltpu.SMEM),
        ),
        scratch_shapes=[
            pltpu.VMEM((B * 8, H), jnp.float32),
            pltpu.SMEM((B,), jnp.float32),
            pltpu.VMEM((1, I), jnp.float32),
        ],
        compiler_params=pltpu.CompilerParams(
            vmem_limit_bytes=110 * 1024 * 1024,
        ),
    )(batch_idx, X, M,
      W1.T, b1.reshape(1, H), W2.T, b2.reshape(1, I),
      wr_x, wr_h, br_p,
      Ws.T[:I], Ws.T[I:], bs.reshape(1, H),
      wt_x, wt_h, bt_p)
    loss = out[0][0]
    ratio = out[1][0]
    return (loss, ratio)


# speedup vs baseline: 92.9185x; 5.9044x over previous
import math

import jax
import jax.numpy as jnp
from jax import lax
from jax.experimental import pallas as pl
from jax.experimental.pallas import tpu as pltpu

H = 512
I = 128
S = 7
B = 8
L = 2048

LOG10_HALF = math.log(10.0) / 2.0
LOG_TAU = [s * LOG10_HALF for s in range(S)]
TAU = [math.exp(v) for v in LOG_TAU]


def _pick_row(blk, rem):
    rowmask = lax.broadcasted_iota(jnp.int32, blk.shape, 0) == rem
    return jnp.sum(jnp.where(rowmask, blk, 0.0), axis=0, keepdims=True)


def _ctgru_batched_kernel(
    perm_ref,
    counts_ref,
    offs_ref,
    nsteps_ref,
    x_ref,
    m_ref,
    w1t_ref,
    b1_ref,
    w2t_ref,
    b2_ref,
    wr_x_ref,
    wr_h_ref,
    br_ref,
    ws_x_ref,
    ws_h_ref,
    bs_ref,
    wt_x_ref,
    wt_h_ref,
    bt_ref,
    loss_ref,
    ratio_ref,
    state_ref,
    lastt_ref,
    acc_ref,
):
    state_ref[...] = jnp.zeros((8 * B, H), jnp.float32)
    acc_ref[...] = jnp.zeros((B, I), jnp.float32)
    for b in range(B):
        lastt_ref[b] = 0.0

    def step(j, carry):
        xs, ms, acts, ots, ivs = [], [], [], [], []
        for b in range(B):
            nb = counts_ref[b]
            pos = jnp.maximum(offs_ref[b] + jnp.minimum(j, nb - 1), 0)
            t = perm_ref[pos]
            tbase = (t // 8) * 8
            trem = t - tbase
            xs.append(_pick_row(x_ref[pl.ds(tbase, 8), :], trem))
            ms.append(_pick_row(m_ref[pl.ds(tbase, 8), :], trem))
            active = j < nb
            acts.append(jnp.full((1, 1), active.astype(jnp.float32)))
            ot = t.astype(jnp.float32)
            lt = lastt_ref[b]
            ots.append(jnp.full((1, 1), ot))
            ivs.append(jnp.full((1, 1), ot - lt))
            lastt_ref[b] = jnp.where(active, ot, lt)
        x8 = jnp.concatenate(xs, axis=0)
        m8 = jnp.concatenate(ms, axis=0)
        act = jnp.concatenate(acts, axis=0)
        iv = jnp.concatenate(ivs, axis=0)
        actb = act > 0.5

        h8 = state_ref[S * 8:(S + 1) * 8, :]

        a = jnp.maximum(
            jnp.dot(h8, w1t_ref[...], preferred_element_type=jnp.float32)
            + b1_ref[...], 0.0)
        p = jnp.dot(a, w2t_ref[...], preferred_element_type=jnp.float32) + b2_ref[...]
        acc_ref[...] += jnp.abs(x8 - p) * m8 * act

        rr = (jnp.dot(x8, wr_x_ref[...], preferred_element_type=jnp.float32)
              + jnp.dot(h8, wr_h_ref[...], preferred_element_type=jnp.float32)
              + br_ref[...])
        q = [-jnp.square(rr[:, s * H:(s + 1) * H] - LOG_TAU[s]) for s in range(S)]
        mx = q[0]
        for s in range(1, S):
            mx = jnp.maximum(mx, q[s])
        e = [jnp.exp(q[s] - mx) for s in range(S)]
        den = e[0]
        for s in range(1, S):
            den = den + e[s]
        hh = [state_ref[s * 8:(s + 1) * 8, :] for s in range(S)]
        rsum = e[0] * hh[0]
        for s in range(1, S):
            rsum += e[s] * hh[s]
        rsum = rsum / den

        h_tilde = jnp.tanh(
            jnp.dot(x8, ws_x_ref[...], preferred_element_type=jnp.float32)
            + jnp.dot(rsum, ws_h_ref[...], preferred_element_type=jnp.float32)
            + bs_ref[...])

        zz = (jnp.dot(x8, wt_x_ref[...], preferred_element_type=jnp.float32)
              + jnp.dot(h8, wt_h_ref[...], preferred_element_type=jnp.float32)
              + bt_ref[...])
        qz = [-jnp.square(zz[:, s * H:(s + 1) * H] - LOG_TAU[s]) for s in range(S)]
        mz = qz[0]
        for s in range(1, S):
            mz = jnp.maximum(mz, qz[s])
        ez = [jnp.exp(qz[s] - mz) for s in range(S)]
        dz = ez[0]
        for s in range(1, S):
            dz = dz + ez[s]

        new_h = hh[0]
        for s in range(1, S):
            new_h = new_h + hh[s]

        for s in range(S):
            z_s = ez[s] / dz
            expf = jnp.exp(-iv / TAU[s])
            new_hh_s = ((1.0 - z_s) * hh[s] + z_s * h_tilde) * expf
            state_ref[s * 8:(s + 1) * 8, :] = jnp.where(actb, new_hh_s, hh[s])
        state_ref[S * 8:(S + 1) * 8, :] = jnp.where(actb, new_h, h8)
        return carry

    lax.fori_loop(0, nsteps_ref[0], step, 0)

    loss = jnp.sum(acc_ref[...])
    tot_m = jnp.sum(m_ref[...])
    loss_ref[0] = loss
    ratio_ref[0] = loss / tot_m


def kernel(obs_times, event_pt, sample_idx, X, M, batch_idx, device, T,
           W1, b1, W2, b2, Wr, br, Ws, bs, Wst, bst):
    perm = jnp.argsort(batch_idx, stable=True).astype(jnp.int32)
    counts = jnp.bincount(batch_idx, length=B).astype(jnp.int32)
    offs = (jnp.cumsum(counts) - counts).astype(jnp.int32)
    nsteps = jnp.max(counts).reshape(1)

    def split_sh(W):
        Wp = W.reshape(H, S, I + H).transpose(2, 1, 0).reshape(I + H, S * H)
        return Wp[:I], Wp[I:]

    wr_x, wr_h = split_sh(Wr)
    wt_x, wt_h = split_sh(Wst)
    br_p = br.reshape(H, S).T.reshape(1, S * H)
    bt_p = bst.reshape(H, S).T.reshape(1, S * H)
    out = pl.pallas_call(
        _ctgru_batched_kernel,
        out_shape=(
            jax.ShapeDtypeStruct((1,), jnp.float32),
            jax.ShapeDtypeStruct((1,), jnp.float32),
        ),
        in_specs=[
            pl.BlockSpec(memory_space=pltpu.SMEM) for _ in range(4)
        ] + [
            pl.BlockSpec(memory_space=pltpu.VMEM) for _ in range(15)
        ],
        out_specs=(
            pl.BlockSpec(memory_space=pltpu.SMEM),
            pl.BlockSpec(memory_space=pltpu.SMEM),
        ),
        scratch_shapes=[
            pltpu.VMEM((8 * B, H), jnp.float32),
            pltpu.SMEM((B,), jnp.float32),
            pltpu.VMEM((B, I), jnp.float32),
        ],
        compiler_params=pltpu.CompilerParams(
            vmem_limit_bytes=110 * 1024 * 1024,
        ),
    )(perm, counts, offs, nsteps, X, M,
      W1.T, b1.reshape(1, H), W2.T, b2.reshape(1, I),
      wr_x, wr_h, br_p,
      Ws.T[:I], Ws.T[I:], bs.reshape(1, H),
      wt_x, wt_h, bt_p)
    loss = out[0][0]
    ratio = out[1][0]
    return (loss, ratio)


# speedup vs baseline: 93.8117x; 1.0096x over previous
import functools
import math

import jax
import jax.numpy as jnp
from jax import lax
from jax.experimental import pallas as pl
from jax.experimental.pallas import tpu as pltpu
from jax.experimental.pallas import tpu_sc as plsc

H = 512
I = 128
S = 7
B = 8
L = 2048

LOG10_HALF = math.log(10.0) / 2.0
LOG_TAU = [s * LOG10_HALF for s in range(S)]
TAU = [math.exp(v) for v in LOG_TAU]


def _pick_row(blk, rem):
    rowmask = lax.broadcasted_iota(jnp.int32, blk.shape, 0) == rem
    return jnp.sum(jnp.where(rowmask, blk, 0.0), axis=0, keepdims=True)


def _routing_sc_kernel(bidx_hbm, perm_hbm, meta_hbm, bidx_v, perm_v, meta_v):
    cid = lax.axis_index("c")
    sid = lax.axis_index("s")

    @pl.when(jnp.logical_and(cid == 0, sid == 0))
    def _():
        pltpu.sync_copy(bidx_hbm, bidx_v)
        iota = lax.broadcasted_iota(jnp.int32, (16,), 0)
        zero = jnp.zeros((16,), jnp.int32)
        zs = jnp.int32(0)

        def count_body(k, cnts):
            v = bidx_v[pl.ds(pl.multiple_of(k * 16, 16), 16)]
            return tuple(cnts[b] + jnp.sum((v == b).astype(jnp.int32))
                         for b in range(B))

        cnts = lax.fori_loop(0, L // 16, count_body, (zs,) * B)
        offs = []
        run = zs
        for b in range(B):
            offs.append(run)
            run = run + cnts[b]

        def scatter_body(k, pos):
            base = k * 16
            v = bidx_v[pl.ds(pl.multiple_of(base, 16), 16)]
            ids = iota + base
            newpos = []
            for b in range(B):
                mask = v == b
                mi = mask.astype(jnp.int32)
                ranks = plsc.cumsum(mi)
                plsc.store_scatter(perm_v, [pos[b] + ranks - 1], ids, mask=mask)
                newpos.append(pos[b] + jnp.sum(mi))
            return tuple(newpos)

        lax.fori_loop(0, L // 16, scatter_body, tuple(offs))

        cnt_lane = zero
        off_lane = zero
        for b in range(B):
            sel = iota == b
            cnt_lane = jnp.where(sel, cnts[b], cnt_lane)
            off_lane = jnp.where(sel, offs[b], off_lane)
        nsteps = cnts[0]
        for b in range(1, B):
            nsteps = jnp.maximum(nsteps, cnts[b])
        meta_v[pl.ds(0, 16)] = cnt_lane
        meta_v[pl.ds(16, 16)] = off_lane
        meta_v[pl.ds(32, 16)] = zero + nsteps
        pltpu.sync_copy(perm_v, perm_hbm)
        pltpu.sync_copy(meta_v, meta_hbm)


def _route_events(batch_idx):
    return pl.kernel(
        _routing_sc_kernel,
        out_type=(
            jax.ShapeDtypeStruct((L,), jnp.int32),
            jax.ShapeDtypeStruct((48,), jnp.int32),
        ),
        mesh=plsc.VectorSubcoreMesh(core_axis_name="c", subcore_axis_name="s"),
        scratch_types=[
            pltpu.VMEM((L,), jnp.int32),
            pltpu.VMEM((L,), jnp.int32),
            pltpu.VMEM((48,), jnp.int32),
        ],
        compiler_params=pltpu.CompilerParams(needs_layout_passes=False),
    )(batch_idx)


def _ctgru_batched_kernel(
    perm_ref,
    counts_ref,
    offs_ref,
    nsteps_ref,
    x_ref,
    m_ref,
    w1t_ref,
    b1_ref,
    w2t_ref,
    b2_ref,
    wr_x_ref,
    wr_h_ref,
    br_ref,
    ws_x_ref,
    ws_h_ref,
    bs_ref,
    wt_x_ref,
    wt_h_ref,
    bt_ref,
    loss_ref,
    ratio_ref,
    state_ref,
    lastt_ref,
    acc_ref,
):
    state_ref[...] = jnp.zeros((8 * B, H), jnp.float32)
    acc_ref[...] = jnp.zeros((B, I), jnp.float32)
    for b in range(B):
        lastt_ref[b] = 0.0

    def step(j, carry):
        xs, ms, acts, ots, ivs = [], [], [], [], []
        for b in range(B):
            nb = counts_ref[b]
            pos = jnp.maximum(offs_ref[b] + jnp.minimum(j, nb - 1), 0)
            t = perm_ref[pos]
            tbase = (t // 8) * 8
            trem = t - tbase
            xs.append(_pick_row(x_ref[pl.ds(tbase, 8), :], trem))
            ms.append(_pick_row(m_ref[pl.ds(tbase, 8), :], trem))
            active = j < nb
            acts.append(jnp.full((1, 1), active.astype(jnp.float32)))
            ot = t.astype(jnp.float32)
            lt = lastt_ref[b]
            ots.append(jnp.full((1, 1), ot))
            ivs.append(jnp.full((1, 1), ot - lt))
            lastt_ref[b] = jnp.where(active, ot, lt)
        x8 = jnp.concatenate(xs, axis=0)
        m8 = jnp.concatenate(ms, axis=0)
        act = jnp.concatenate(acts, axis=0)
        iv = jnp.concatenate(ivs, axis=0)
        actb = act > 0.5

        h8 = state_ref[S * 8:(S + 1) * 8, :]

        a = jnp.maximum(
            jnp.dot(h8, w1t_ref[...], preferred_element_type=jnp.float32)
            + b1_ref[...], 0.0)
        p = jnp.dot(a, w2t_ref[...], preferred_element_type=jnp.float32) + b2_ref[...]
        acc_ref[...] += jnp.abs(x8 - p) * m8 * act

        rr = (jnp.dot(x8, wr_x_ref[...], preferred_element_type=jnp.float32)
              + jnp.dot(h8, wr_h_ref[...], preferred_element_type=jnp.float32)
              + br_ref[...])
        q = [-jnp.square(rr[:, s * H:(s + 1) * H] - LOG_TAU[s]) for s in range(S)]
        mx = q[0]
        for s in range(1, S):
            mx = jnp.maximum(mx, q[s])
        e = [jnp.exp(q[s] - mx) for s in range(S)]
        den = e[0]
        for s in range(1, S):
            den = den + e[s]
        hh = [state_ref[s * 8:(s + 1) * 8, :] for s in range(S)]
        rsum = e[0] * hh[0]
        for s in range(1, S):
            rsum += e[s] * hh[s]
        rsum = rsum / den

        h_tilde = jnp.tanh(
            jnp.dot(x8, ws_x_ref[...], preferred_element_type=jnp.float32)
            + jnp.dot(rsum, ws_h_ref[...], preferred_element_type=jnp.float32)
            + bs_ref[...])

        zz = (jnp.dot(x8, wt_x_ref[...], preferred_element_type=jnp.float32)
              + jnp.dot(h8, wt_h_ref[...], preferred_element_type=jnp.float32)
              + bt_ref[...])
        qz = [-jnp.square(zz[:, s * H:(s + 1) * H] - LOG_TAU[s]) for s in range(S)]
        mz = qz[0]
        for s in range(1, S):
            mz = jnp.maximum(mz, qz[s])
        ez = [jnp.exp(qz[s] - mz) for s in range(S)]
        dz = ez[0]
        for s in range(1, S):
            dz = dz + ez[s]

        new_h = hh[0]
        for s in range(1, S):
            new_h = new_h + hh[s]

        for s in range(S):
            z_s = ez[s] / dz
            expf = jnp.exp(-iv / TAU[s])
            new_hh_s = ((1.0 - z_s) * hh[s] + z_s * h_tilde) * expf
            state_ref[s * 8:(s + 1) * 8, :] = jnp.where(actb, new_hh_s, hh[s])
        state_ref[S * 8:(S + 1) * 8, :] = jnp.where(actb, new_h, h8)
        return carry

    lax.fori_loop(0, nsteps_ref[0], step, 0)

    loss = jnp.sum(acc_ref[...])
    tot_m = jnp.sum(m_ref[...])
    loss_ref[0] = loss
    ratio_ref[0] = loss / tot_m


def kernel(obs_times, event_pt, sample_idx, X, M, batch_idx, device, T,
           W1, b1, W2, b2, Wr, br, Ws, bs, Wst, bst):
    perm, meta = _route_events(batch_idx)
    counts = meta[0:B]
    offs = meta[16:16 + B]
    nsteps = meta[32:33]

    def split_sh(W):
        Wp = W.reshape(H, S, I + H).transpose(2, 1, 0).reshape(I + H, S * H)
        return Wp[:I], Wp[I:]

    wr_x, wr_h = split_sh(Wr)
    wt_x, wt_h = split_sh(Wst)
    br_p = br.reshape(H, S).T.reshape(1, S * H)
    bt_p = bst.reshape(H, S).T.reshape(1, S * H)
    out = pl.pallas_call(
        _ctgru_batched_kernel,
        out_shape=(
            jax.ShapeDtypeStruct((1,), jnp.float32),
            jax.ShapeDtypeStruct((1,), jnp.float32),
        ),
        in_specs=[
            pl.BlockSpec(memory_space=pltpu.SMEM) for _ in range(4)
        ] + [
            pl.BlockSpec(memory_space=pltpu.VMEM) for _ in range(15)
        ],
        out_specs=(
            pl.BlockSpec(memory_space=pltpu.SMEM),
            pl.BlockSpec(memory_space=pltpu.SMEM),
        ),
        scratch_shapes=[
            pltpu.VMEM((8 * B, H), jnp.float32),
            pltpu.SMEM((B,), jnp.float32),
            pltpu.VMEM((B, I), jnp.float32),
        ],
        compiler_params=pltpu.CompilerParams(
            vmem_limit_bytes=110 * 1024 * 1024,
        ),
    )(perm, counts, offs, nsteps, X, M,
      W1.T, b1.reshape(1, H), W2.T, b2.reshape(1, I),
      wr_x, wr_h, br_p,
      Ws.T[:I], Ws.T[I:], bs.reshape(1, H),
      wt_x, wt_h, bt_p)
    loss = out[0][0]
    ratio = out[1][0]
    return (loss, ratio)


# speedup vs baseline: 95.3540x; 1.0164x over previous
import functools
import math

import jax
import jax.numpy as jnp
from jax import lax
from jax.experimental import pallas as pl
from jax.experimental.pallas import tpu as pltpu
from jax.experimental.pallas import tpu_sc as plsc

H = 512
I = 128
S = 7
B = 8
L = 2048

LOG10_HALF = math.log(10.0) / 2.0
LOG_TAU = [s * LOG10_HALF for s in range(S)]
TAU = [math.exp(v) for v in LOG_TAU]


def _pick_row(blk, rem):
    rowmask = lax.broadcasted_iota(jnp.int32, blk.shape, 0) == rem
    return jnp.sum(jnp.where(rowmask, blk, 0.0), axis=0, keepdims=True)


def _routing_sc_kernel(bidx_hbm, perm_hbm, meta_hbm, bidx_v, perm_v, meta_v):
    cid = lax.axis_index("c")
    sid = lax.axis_index("s")

    @pl.when(jnp.logical_and(cid == 0, sid == 0))
    def _():
        pltpu.sync_copy(bidx_hbm, bidx_v)
        iota = lax.broadcasted_iota(jnp.int32, (16,), 0)
        zero = jnp.zeros((16,), jnp.int32)
        zs = jnp.int32(0)

        def count_body(k, cnts):
            v = bidx_v[pl.ds(pl.multiple_of(k * 16, 16), 16)]
            return tuple(cnts[b] + jnp.sum((v == b).astype(jnp.int32))
                         for b in range(B))

        cnts = lax.fori_loop(0, L // 16, count_body, (zs,) * B)
        offs = []
        run = zs
        for b in range(B):
            offs.append(run)
            run = run + cnts[b]

        def scatter_body(k, pos):
            base = k * 16
            v = bidx_v[pl.ds(pl.multiple_of(base, 16), 16)]
            ids = iota + base
            newpos = []
            for b in range(B):
                mask = v == b
                mi = mask.astype(jnp.int32)
                ranks = plsc.cumsum(mi)
                plsc.store_scatter(perm_v, [pos[b] + ranks - 1], ids, mask=mask)
                newpos.append(pos[b] + jnp.sum(mi))
            return tuple(newpos)

        lax.fori_loop(0, L // 16, scatter_body, tuple(offs))

        cnt_lane = zero
        off_lane = zero
        for b in range(B):
            sel = iota == b
            cnt_lane = jnp.where(sel, cnts[b], cnt_lane)
            off_lane = jnp.where(sel, offs[b], off_lane)
        nsteps = cnts[0]
        for b in range(1, B):
            nsteps = jnp.maximum(nsteps, cnts[b])
        meta_v[pl.ds(0, 16)] = cnt_lane
        meta_v[pl.ds(16, 16)] = off_lane
        meta_v[pl.ds(32, 16)] = zero + nsteps
        pltpu.sync_copy(perm_v, perm_hbm)
        pltpu.sync_copy(meta_v, meta_hbm)


def _route_events(batch_idx):
    return pl.kernel(
        _routing_sc_kernel,
        out_type=(
            jax.ShapeDtypeStruct((L,), jnp.int32),
            jax.ShapeDtypeStruct((48,), jnp.int32),
        ),
        mesh=plsc.VectorSubcoreMesh(core_axis_name="c", subcore_axis_name="s"),
        scratch_types=[
            pltpu.VMEM((L,), jnp.int32),
            pltpu.VMEM((L,), jnp.int32),
            pltpu.VMEM((48,), jnp.int32),
        ],
        compiler_params=pltpu.CompilerParams(needs_layout_passes=False),
    )(batch_idx)


def _ctgru_batched_kernel(
    perm_ref,
    counts_ref,
    offs_ref,
    nsteps_ref,
    x_ref,
    m_ref,
    w1t_ref,
    b1_ref,
    w2t_ref,
    b2_ref,
    wr_x_ref,
    wr_h_ref,
    br_ref,
    ws_x_ref,
    ws_h_ref,
    bs_ref,
    wt_x_ref,
    wt_h_ref,
    bt_ref,
    loss_ref,
    ratio_ref,
    state_ref,
    lastt_ref,
    acc_ref,
):
    state_ref[...] = jnp.zeros((8 * B, H), jnp.float32)
    acc_ref[...] = jnp.zeros((B, I), jnp.float32)
    for b in range(B):
        lastt_ref[b] = 0.0

    def step(j, carry):
        xs, ms, acts, ots, ivs = [], [], [], [], []
        for b in range(B):
            nb = counts_ref[b]
            pos = jnp.maximum(offs_ref[b] + jnp.minimum(j, nb - 1), 0)
            t = perm_ref[pos]
            tbase = (t // 8) * 8
            trem = t - tbase
            xs.append(_pick_row(x_ref[pl.ds(tbase, 8), :], trem))
            ms.append(_pick_row(m_ref[pl.ds(tbase, 8), :], trem))
            active = j < nb
            acts.append(jnp.full((1, 1), active.astype(jnp.float32)))
            ot = t.astype(jnp.float32)
            lt = lastt_ref[b]
            ots.append(jnp.full((1, 1), ot))
            ivs.append(jnp.full((1, 1), ot - lt))
            lastt_ref[b] = jnp.where(active, ot, lt)
        x8 = jnp.concatenate(xs, axis=0)
        m8 = jnp.concatenate(ms, axis=0)
        act = jnp.concatenate(acts, axis=0)
        iv = jnp.concatenate(ivs, axis=0)
        actb = act > 0.5

        h8 = state_ref[S * 8:(S + 1) * 8, :]
        h8b = h8.astype(jnp.bfloat16)
        x8b = x8.astype(jnp.bfloat16)

        a = jnp.maximum(
            jnp.dot(h8b, w1t_ref[...], preferred_element_type=jnp.float32)
            + b1_ref[...], 0.0)
        p = jnp.dot(a.astype(jnp.bfloat16), w2t_ref[...], preferred_element_type=jnp.float32) + b2_ref[...]
        acc_ref[...] += jnp.abs(x8 - p) * m8 * act

        rr = (jnp.dot(x8b, wr_x_ref[...], preferred_element_type=jnp.float32)
              + jnp.dot(h8b, wr_h_ref[...], preferred_element_type=jnp.float32)
              + br_ref[...])
        q = [-jnp.square(rr[:, s * H:(s + 1) * H] - LOG_TAU[s]) for s in range(S)]
        mx = q[0]
        for s in range(1, S):
            mx = jnp.maximum(mx, q[s])
        e = [jnp.exp(q[s] - mx) for s in range(S)]
        den = e[0]
        for s in range(1, S):
            den = den + e[s]
        hh = [state_ref[s * 8:(s + 1) * 8, :] for s in range(S)]
        rsum = e[0] * hh[0]
        for s in range(1, S):
            rsum += e[s] * hh[s]
        rsum = rsum / den

        h_tilde = jnp.tanh(
            jnp.dot(x8b, ws_x_ref[...], preferred_element_type=jnp.float32)
            + jnp.dot(rsum.astype(jnp.bfloat16), ws_h_ref[...], preferred_element_type=jnp.float32)
            + bs_ref[...])

        zz = (jnp.dot(x8b, wt_x_ref[...], preferred_element_type=jnp.float32)
              + jnp.dot(h8b, wt_h_ref[...], preferred_element_type=jnp.float32)
              + bt_ref[...])
        qz = [-jnp.square(zz[:, s * H:(s + 1) * H] - LOG_TAU[s]) for s in range(S)]
        mz = qz[0]
        for s in range(1, S):
            mz = jnp.maximum(mz, qz[s])
        ez = [jnp.exp(qz[s] - mz) for s in range(S)]
        dz = ez[0]
        for s in range(1, S):
            dz = dz + ez[s]

        new_h = hh[0]
        for s in range(1, S):
            new_h = new_h + hh[s]

        for s in range(S):
            z_s = ez[s] / dz
            expf = jnp.exp(-iv / TAU[s])
            new_hh_s = ((1.0 - z_s) * hh[s] + z_s * h_tilde) * expf
            state_ref[s * 8:(s + 1) * 8, :] = jnp.where(actb, new_hh_s, hh[s])
        state_ref[S * 8:(S + 1) * 8, :] = jnp.where(actb, new_h, h8)
        return carry

    lax.fori_loop(0, nsteps_ref[0], step, 0)

    loss = jnp.sum(acc_ref[...])
    tot_m = jnp.sum(m_ref[...])
    loss_ref[0] = loss
    ratio_ref[0] = loss / tot_m


def kernel(obs_times, event_pt, sample_idx, X, M, batch_idx, device, T,
           W1, b1, W2, b2, Wr, br, Ws, bs, Wst, bst):
    perm, meta = _route_events(batch_idx)
    counts = meta[0:B]
    offs = meta[16:16 + B]
    nsteps = meta[32:33]

    def split_sh(W):
        Wp = W.reshape(H, S, I + H).transpose(2, 1, 0).reshape(I + H, S * H)
        return Wp[:I], Wp[I:]

    bf = lambda w: w.astype(jnp.bfloat16)
    wr_x, wr_h = split_sh(Wr)
    wt_x, wt_h = split_sh(Wst)
    br_p = br.reshape(H, S).T.reshape(1, S * H)
    bt_p = bst.reshape(H, S).T.reshape(1, S * H)
    out = pl.pallas_call(
        _ctgru_batched_kernel,
        out_shape=(
            jax.ShapeDtypeStruct((1,), jnp.float32),
            jax.ShapeDtypeStruct((1,), jnp.float32),
        ),
        in_specs=[
            pl.BlockSpec(memory_space=pltpu.SMEM) for _ in range(4)
        ] + [
            pl.BlockSpec(memory_space=pltpu.VMEM) for _ in range(15)
        ],
        out_specs=(
            pl.BlockSpec(memory_space=pltpu.SMEM),
            pl.BlockSpec(memory_space=pltpu.SMEM),
        ),
        scratch_shapes=[
            pltpu.VMEM((8 * B, H), jnp.float32),
            pltpu.SMEM((B,), jnp.float32),
            pltpu.VMEM((B, I), jnp.float32),
        ],
        compiler_params=pltpu.CompilerParams(
            vmem_limit_bytes=110 * 1024 * 1024,
        ),
    )(perm, counts, offs, nsteps, X, M,
      bf(W1.T), b1.reshape(1, H), bf(W2.T), b2.reshape(1, I),
      bf(wr_x), bf(wr_h), br_p,
      bf(Ws.T[:I]), bf(Ws.T[I:]), bs.reshape(1, H),
      bf(wt_x), bf(wt_h), bt_p)
    loss = out[0][0]
    ratio = out[1][0]
    return (loss, ratio)


# speedup vs baseline: 100.7848x; 1.0570x over previous
import functools
import math

import jax
import jax.numpy as jnp
from jax import lax
from jax.experimental import pallas as pl
from jax.experimental.pallas import tpu as pltpu
from jax.experimental.pallas import tpu_sc as plsc

H = 512
I = 128
S = 7
B = 8
L = 2048

LOG10_HALF = math.log(10.0) / 2.0
LOG_TAU = [s * LOG10_HALF for s in range(S)]
TAU = [math.exp(v) for v in LOG_TAU]


def _pick_row(blk, rem):
    rowmask = lax.broadcasted_iota(jnp.int32, blk.shape, 0) == rem
    return jnp.sum(jnp.where(rowmask, blk, 0.0), axis=0, keepdims=True)


def _routing_sc_kernel(bidx_hbm, perm_hbm, meta_hbm, bidx_v, perm_v, meta_v):
    cid = lax.axis_index("c")
    sid = lax.axis_index("s")

    @pl.when(jnp.logical_and(cid == 0, sid == 0))
    def _():
        pltpu.sync_copy(bidx_hbm, bidx_v)
        iota = lax.broadcasted_iota(jnp.int32, (16,), 0)
        zero = jnp.zeros((16,), jnp.int32)
        zs = jnp.int32(0)

        def count_body(k, cnts):
            v = bidx_v[pl.ds(pl.multiple_of(k * 16, 16), 16)]
            return tuple(cnts[b] + jnp.sum((v == b).astype(jnp.int32))
                         for b in range(B))

        cnts = lax.fori_loop(0, L // 16, count_body, (zs,) * B)
        offs = []
        run = zs
        for b in range(B):
            offs.append(run)
            run = run + cnts[b]

        def scatter_body(k, pos):
            base = k * 16
            v = bidx_v[pl.ds(pl.multiple_of(base, 16), 16)]
            ids = iota + base
            newpos = []
            for b in range(B):
                mask = v == b
                mi = mask.astype(jnp.int32)
                ranks = plsc.cumsum(mi)
                plsc.store_scatter(perm_v, [pos[b] + ranks - 1], ids, mask=mask)
                newpos.append(pos[b] + jnp.sum(mi))
            return tuple(newpos)

        lax.fori_loop(0, L // 16, scatter_body, tuple(offs))

        cnt_lane = zero
        off_lane = zero
        for b in range(B):
            sel = iota == b
            cnt_lane = jnp.where(sel, cnts[b], cnt_lane)
            off_lane = jnp.where(sel, offs[b], off_lane)
        nsteps = cnts[0]
        for b in range(1, B):
            nsteps = jnp.maximum(nsteps, cnts[b])
        meta_v[pl.ds(0, 16)] = cnt_lane
        meta_v[pl.ds(16, 16)] = off_lane
        meta_v[pl.ds(32, 16)] = zero + nsteps
        pltpu.sync_copy(perm_v, perm_hbm)
        pltpu.sync_copy(meta_v, meta_hbm)


def _route_events(batch_idx):
    return pl.kernel(
        _routing_sc_kernel,
        out_type=(
            jax.ShapeDtypeStruct((L,), jnp.int32),
            jax.ShapeDtypeStruct((48,), jnp.int32),
        ),
        mesh=plsc.VectorSubcoreMesh(core_axis_name="c", subcore_axis_name="s"),
        scratch_types=[
            pltpu.VMEM((L,), jnp.int32),
            pltpu.VMEM((L,), jnp.int32),
            pltpu.VMEM((48,), jnp.int32),
        ],
        compiler_params=pltpu.CompilerParams(needs_layout_passes=False),
    )(batch_idx)


def _ctgru_batched_kernel(
    perm_ref,
    counts_ref,
    offs_ref,
    nsteps_ref,
    x_ref,
    m_ref,
    w1t_ref,
    b1_ref,
    w2t_ref,
    b2_ref,
    wxc_ref,
    whc_ref,
    bc_ref,
    ws_x_ref,
    ws_h_ref,
    bs_ref,
    loss_ref,
    ratio_ref,
    state_ref,
    lastt_ref,
    acc_ref,
):
    state_ref[...] = jnp.zeros((8 * B, H), jnp.float32)
    acc_ref[...] = jnp.zeros((B, I), jnp.float32)
    for b in range(B):
        lastt_ref[b] = 0.0

    def step(j, carry):
        xs, ms, acts, ots, ivs = [], [], [], [], []
        for b in range(B):
            nb = counts_ref[b]
            pos = jnp.maximum(offs_ref[b] + jnp.minimum(j, nb - 1), 0)
            t = perm_ref[pos]
            tbase = (t // 8) * 8
            trem = t - tbase
            xs.append(_pick_row(x_ref[pl.ds(tbase, 8), :], trem))
            ms.append(_pick_row(m_ref[pl.ds(tbase, 8), :], trem))
            active = j < nb
            acts.append(jnp.full((1, 1), active.astype(jnp.float32)))
            ot = t.astype(jnp.float32)
            lt = lastt_ref[b]
            ots.append(jnp.full((1, 1), ot))
            ivs.append(jnp.full((1, 1), ot - lt))
            lastt_ref[b] = jnp.where(active, ot, lt)
        x8 = jnp.concatenate(xs, axis=0)
        m8 = jnp.concatenate(ms, axis=0)
        act = jnp.concatenate(acts, axis=0)
        iv = jnp.concatenate(ivs, axis=0)
        actb = act > 0.5

        h8 = state_ref[S * 8:(S + 1) * 8, :]
        h8b = h8.astype(jnp.bfloat16)
        x8b = x8.astype(jnp.bfloat16)

        rz = (jnp.dot(x8b, wxc_ref[...], preferred_element_type=jnp.float32)
              + jnp.dot(h8b, whc_ref[...], preferred_element_type=jnp.float32)
              + bc_ref[...])

        a = jnp.maximum(
            jnp.dot(h8b, w1t_ref[...], preferred_element_type=jnp.float32)
            + b1_ref[...], 0.0)
        p = jnp.dot(a.astype(jnp.bfloat16), w2t_ref[...], preferred_element_type=jnp.float32) + b2_ref[...]
        acc_ref[...] += jnp.abs(x8 - p) * m8 * act

        q = [-jnp.square(rz[:, s * H:(s + 1) * H] - LOG_TAU[s]) for s in range(S)]
        mx = q[0]
        for s in range(1, S):
            mx = jnp.maximum(mx, q[s])
        e = [jnp.exp(q[s] - mx) for s in range(S)]
        den = e[0]
        for s in range(1, S):
            den = den + e[s]
        hh = [state_ref[s * 8:(s + 1) * 8, :] for s in range(S)]
        rsum = e[0] * hh[0]
        for s in range(1, S):
            rsum += e[s] * hh[s]
        rsum = rsum / den

        h_tilde = jnp.tanh(
            jnp.dot(x8b, ws_x_ref[...], preferred_element_type=jnp.float32)
            + jnp.dot(rsum.astype(jnp.bfloat16), ws_h_ref[...], preferred_element_type=jnp.float32)
            + bs_ref[...])

        Z0 = S * H
        qz = [-jnp.square(rz[:, Z0 + s * H:Z0 + (s + 1) * H] - LOG_TAU[s]) for s in range(S)]
        mz = qz[0]
        for s in range(1, S):
            mz = jnp.maximum(mz, qz[s])
        ez = [jnp.exp(qz[s] - mz) for s in range(S)]
        dz = ez[0]
        for s in range(1, S):
            dz = dz + ez[s]

        new_h = hh[0]
        for s in range(1, S):
            new_h = new_h + hh[s]

        for s in range(S):
            z_s = ez[s] / dz
            expf = jnp.exp(-iv / TAU[s])
            new_hh_s = ((1.0 - z_s) * hh[s] + z_s * h_tilde) * expf
            state_ref[s * 8:(s + 1) * 8, :] = jnp.where(actb, new_hh_s, hh[s])
        state_ref[S * 8:(S + 1) * 8, :] = jnp.where(actb, new_h, h8)
        return carry

    lax.fori_loop(0, nsteps_ref[0], step, 0)

    loss = jnp.sum(acc_ref[...])
    tot_m = jnp.sum(m_ref[...])
    loss_ref[0] = loss
    ratio_ref[0] = loss / tot_m


def kernel(obs_times, event_pt, sample_idx, X, M, batch_idx, device, T,
           W1, b1, W2, b2, Wr, br, Ws, bs, Wst, bst):
    perm, meta = _route_events(batch_idx)
    counts = meta[0:B]
    offs = meta[16:16 + B]
    nsteps = meta[32:33]

    def split_sh(W):
        Wp = W.reshape(H, S, I + H).transpose(2, 1, 0).reshape(I + H, S * H)
        return Wp[:I], Wp[I:]

    bf = lambda w: w.astype(jnp.bfloat16)
    wr_x, wr_h = split_sh(Wr)
    wt_x, wt_h = split_sh(Wst)
    wxc = jnp.concatenate([wr_x, wt_x], axis=1)
    whc = jnp.concatenate([wr_h, wt_h], axis=1)
    br_p = br.reshape(H, S).T.reshape(1, S * H)
    bt_p = bst.reshape(H, S).T.reshape(1, S * H)
    bc = jnp.concatenate([br_p, bt_p], axis=1)
    out = pl.pallas_call(
        _ctgru_batched_kernel,
        out_shape=(
            jax.ShapeDtypeStruct((1,), jnp.float32),
            jax.ShapeDtypeStruct((1,), jnp.float32),
        ),
        in_specs=[
            pl.BlockSpec(memory_space=pltpu.SMEM) for _ in range(4)
        ] + [
            pl.BlockSpec(memory_space=pltpu.VMEM) for _ in range(12)
        ],
        out_specs=(
            pl.BlockSpec(memory_space=pltpu.SMEM),
            pl.BlockSpec(memory_space=pltpu.SMEM),
        ),
        scratch_shapes=[
            pltpu.VMEM((8 * B, H), jnp.float32),
            pltpu.SMEM((B,), jnp.float32),
            pltpu.VMEM((B, I), jnp.float32),
        ],
        compiler_params=pltpu.CompilerParams(
            vmem_limit_bytes=110 * 1024 * 1024,
        ),
    )(perm, counts, offs, nsteps, X, M,
      bf(W1.T), b1.reshape(1, H), bf(W2.T), b2.reshape(1, I),
      bf(wxc), bf(whc), bc,
      bf(Ws.T[:I]), bf(Ws.T[I:]), bs.reshape(1, H))
    loss = out[0][0]
    ratio = out[1][0]
    return (loss, ratio)
